# baseline (speedup 1.0000x reference)
import functools

import jax
import jax.numpy as jnp
from jax import lax
from jax.experimental import pallas as pl
from jax.experimental.pallas import tpu as pltpu
from jax.experimental.pallas import tpu_sc as plsc

DIM = 64
CH = 128


@functools.cache
def _make_gather(n_q: int, n_t: int, dim: int):
    info = plsc.get_sparse_core_info()
    NC, NS = info.num_cores, info.num_subcores
    NW = NC * NS
    assert n_q % (NW * CH) == 0 and n_t % (NW * CH) == 0
    q_ch = n_q // (NW * CH)
    t_ch = n_t // (NW * CH)
    assert q_ch % 2 == 0 and t_ch % 2 == 0
    q_per = q_ch * CH
    t_per = t_ch * CH

    mesh = plsc.VectorSubcoreMesh(core_axis_name="c", subcore_axis_name="s")

    @functools.partial(
        pl.kernel,
        out_type=(
            jax.ShapeDtypeStruct((n_q, dim), jnp.float32),
            jax.ShapeDtypeStruct((n_t, dim), jnp.float32),
        ),
        mesh=mesh,
        compiler_params=pltpu.CompilerParams(use_tc_tiling_on_sc=False),
        scratch_types=[
            pltpu.VMEM((q_per + t_per,), jnp.int32),
            pltpu.VMEM((2, CH, dim), jnp.float32),
            pltpu.SemaphoreType.DMA,
            pltpu.SemaphoreType.DMA,
        ],
    )
    def gather_kernel(table_hbm, q_hbm, t_hbm, out_q, out_t,
                      idx_v, rows_v, sem0, sem1):
        c = lax.axis_index("c")
        s = lax.axis_index("s")
        wid = s * NC + c

        pltpu.sync_copy(q_hbm.at[pl.ds(wid * q_per, q_per)],
                        idx_v.at[pl.ds(0, q_per)])
        pltpu.sync_copy(t_hbm.at[pl.ds(wid * t_per, t_per)],
                        idx_v.at[pl.ds(q_per, t_per)])

        sems = (sem0, sem1)

        def run(out_ref, out_base, off0, n_ch):
            def start(j, buf):
                pltpu.async_copy(
                    table_hbm.at[idx_v.at[pl.ds(off0 + j * CH, CH)]],
                    rows_v.at[buf], sems[buf])

            def drain(j, buf):
                pltpu.make_async_copy(
                    table_hbm.at[idx_v.at[pl.ds(off0 + j * CH, CH)]],
                    rows_v.at[buf], sems[buf]).wait()
                pltpu.sync_copy(rows_v.at[buf],
                                out_ref.at[pl.ds(out_base + j * CH, CH)])

            start(0, 0)

            def step(i, _):
                j0 = i * 2
                start(j0 + 1, 1)
                drain(j0, 0)
                start(j0 + 2, 0)
                drain(j0 + 1, 1)
                return 0

            lax.fori_loop(0, n_ch // 2 - 1, step, 0)
            j0 = n_ch - 2
            start(j0 + 1, 1)
            drain(j0, 0)
            drain(j0 + 1, 1)

        run(out_q, wid * q_per, 0, q_ch)
        run(out_t, wid * t_per, q_per, t_ch)

    return gather_kernel


def kernel(table, query, title):
    n_q = query.size
    n_t = title.size
    q = query.astype(jnp.int32).reshape(n_q)
    t = title.astype(jnp.int32).reshape(n_t)
    fn = _make_gather(n_q, n_t, table.shape[1])
    out_q, out_t = fn(table, q, t)
    return (out_q.reshape(*query.shape, table.shape[1]),
            out_t.reshape(*title.shape, table.shape[1]))


# speedup vs baseline: 1.0135x; 1.0135x over previous
import functools

import jax
import jax.numpy as jnp
from jax import lax
from jax.experimental import pallas as pl
from jax.experimental.pallas import tpu as pltpu
from jax.experimental.pallas import tpu_sc as plsc

CH = 128


@functools.cache
def _make_gather(n_q: int, n_t: int, dim: int):
    info = plsc.get_sparse_core_info()
    NC, NS = info.num_cores, info.num_subcores
    NW = NC * NS
    n_all = n_q + n_t
    assert n_q % CH == 0 and n_all % (NW * CH) == 0
    n_ch = n_all // (NW * CH)
    per = n_ch * CH
    q_chunks = n_q // CH
    NBUF = 10
    assert n_ch % NBUF == 0
    ngroups = n_ch // NBUF

    mesh = plsc.VectorSubcoreMesh(core_axis_name="c", subcore_axis_name="s")

    @functools.partial(
        pl.kernel,
        out_type=(
            jax.ShapeDtypeStruct((n_q, dim), jnp.float32),
            jax.ShapeDtypeStruct((n_t, dim), jnp.float32),
        ),
        mesh=mesh,
        compiler_params=pltpu.CompilerParams(use_tc_tiling_on_sc=False),
        scratch_types=[
            pltpu.VMEM((per,), jnp.int32),
            pltpu.VMEM((NBUF, CH, dim), jnp.float32),
            pltpu.SemaphoreType.DMA((NBUF,)),
            pltpu.SemaphoreType.DMA((NBUF,)),
        ],
    )
    def gather_kernel(table_hbm, idx_hbm, out_q, out_t,
                      idx_v, rows_v, gsem, wsem):
        c = lax.axis_index("c")
        s = lax.axis_index("s")
        wid = s * NC + c

        pltpu.sync_copy(idx_hbm.at[pl.ds(wid * per, per)], idx_v)

        def gstart(b, j):
            pltpu.async_copy(table_hbm.at[idx_v.at[pl.ds(j * CH, CH)]],
                             rows_v.at[b], gsem.at[b])

        def gwait(b):
            pltpu.make_async_copy(table_hbm.at[idx_v.at[pl.ds(0, CH)]],
                                  rows_v.at[b], gsem.at[b]).wait()

        def wstart(b, j):
            g = wid * n_ch + j

            @pl.when(g < q_chunks)
            def _():
                pltpu.async_copy(rows_v.at[b],
                                 out_q.at[pl.ds(g * CH, CH)], wsem.at[b])

            @pl.when(g >= q_chunks)
            def _():
                pltpu.async_copy(rows_v.at[b],
                                 out_t.at[pl.ds((g - q_chunks) * CH, CH)],
                                 wsem.at[b])

        def wwait(b):
            pltpu.make_async_copy(rows_v.at[b],
                                  out_q.at[pl.ds(0, CH)], wsem.at[b]).wait()

        for b in range(NBUF):
            gstart(b, b)

        def group(i, _):
            j0 = i * NBUF
            for b in range(NBUF):
                gwait(b)
                wstart(b, j0 + b)
            for b in range(NBUF):
                wwait(b)
                gstart(b, j0 + NBUF + b)
            return 0

        lax.fori_loop(0, ngroups - 1, group, 0)
        j0 = (ngroups - 1) * NBUF
        for b in range(NBUF):
            gwait(b)
            wstart(b, j0 + b)
        for b in range(NBUF):
            wwait(b)

    return gather_kernel


def kernel(table, query, title):
    n_q = query.size
    n_t = title.size
    idx_all = jnp.concatenate([
        query.astype(jnp.int32).reshape(n_q),
        title.astype(jnp.int32).reshape(n_t),
    ])
    fn = _make_gather(n_q, n_t, table.shape[1])
    out_q, out_t = fn(table, idx_all)
    return (out_q.reshape(*query.shape, table.shape[1]),
            out_t.reshape(*title.shape, table.shape[1]))
